# Initial kernel scaffold; baseline (speedup 1.0000x reference)
import jax
import jax.numpy as jnp
from jax.experimental import pallas as pl


def kernel(combined, gate_w, Wi, bi, bn1_g, bn1_b, Wh, bh, bn2_g, bn2_b, Wo, bo):
    raise NotImplementedError("write your pallas kernel here")


# speedup vs baseline: 1.3478x; 1.3478x over previous
import functools

import jax
import jax.numpy as jnp
from jax.experimental import pallas as pl

E = 16
TOPK = 8
D_IN = 256
D_HID = 128
D_OUT = 256
N_TOK = 16384
EPS = 1e-5

BLK = 1024


def _dot_t(a, b):
    return jax.lax.dot_general(
        a, b, dimension_numbers=(((1,), (1,)), ((), ())),
        preferred_element_type=jnp.float32)


def _moe_kernel(x_ref, gw_ref, wi_ref, bi_ref, g1_ref, b1_ref,
                wh_ref, bh_ref, g2_ref, b2_ref, wo_ref, bo_ref, out_ref):
    x = x_ref[:]

    logits = _dot_t(x, gw_ref[:])
    m = jnp.max(logits, axis=-1, keepdims=True)
    ex = jnp.exp(logits - m)
    s = ex / jnp.sum(ex, axis=-1, keepdims=True)

    col = jax.lax.broadcasted_iota(jnp.int32, (BLK, E), 1)
    rank = jnp.zeros((BLK, E), dtype=jnp.int32)
    for j in range(E):
        sj = s[:, j:j + 1]
        rank = rank + jnp.where(sj > s, 1, 0)
        rank = rank + jnp.where((sj == s) & (j < col), 1, 0)
    sel = rank < TOPK
    w = jnp.where(sel, s, 0.0)
    w = w / (jnp.sum(w, axis=-1, keepdims=True) + 1e-20)

    bn_c = 1.0 / jnp.sqrt(1.0 + EPS)

    def expert(e):
        h = jnp.maximum(_dot_t(x, wi_ref[e]) + bi_ref[e:e + 1, :], 0.0)
        h = h * (g1_ref[e:e + 1, :] * bn_c) + b1_ref[e:e + 1, :]
        h2 = jnp.maximum(_dot_t(h, wh_ref[e]) + bh_ref[e:e + 1, :], 0.0)
        h2 = h2 * (g2_ref[e:e + 1, :] * bn_c) + b2_ref[e:e + 1, :]
        return jax.nn.sigmoid(_dot_t(h2, wo_ref[e]) + bo_ref[e:e + 1, :])

    acc = expert(E)
    for e in range(E):
        acc = acc + w[:, e:e + 1] * expert(e)
    out_ref[:] = acc


@jax.jit
def kernel(combined, gate_w, Wi, bi, bn1_g, bn1_b, Wh, bh, bn2_g, bn2_b, Wo, bo):
    nall = E + 1
    full = lambda shape: pl.BlockSpec(shape, lambda i: (0,) * len(shape))
    grid = N_TOK // BLK
    return pl.pallas_call(
        _moe_kernel,
        grid=(grid,),
        in_specs=[
            pl.BlockSpec((BLK, D_IN), lambda i: (i, 0)),
            full((E, D_IN)),
            full((nall, D_HID, D_IN)),
            full((nall, D_HID)),
            full((nall, D_HID)),
            full((nall, D_HID)),
            full((nall, D_HID, D_HID)),
            full((nall, D_HID)),
            full((nall, D_HID)),
            full((nall, D_HID)),
            full((nall, D_OUT, D_HID)),
            full((nall, D_OUT)),
        ],
        out_specs=pl.BlockSpec((BLK, D_OUT), lambda i: (i, 0)),
        out_shape=jax.ShapeDtypeStruct((N_TOK, D_OUT), jnp.float32),
    )(combined, gate_w, Wi, bi, bn1_g, bn1_b, Wh, bh, bn2_g, bn2_b, Wo, bo)
